# Initial kernel scaffold; baseline (speedup 1.0000x reference)
import jax
import jax.numpy as jnp
from jax.experimental import pallas as pl


def kernel(x, edge_index, embed, W, W_ih, W_hh, b_ih, b_hh, gate_w, gate_b):
    raise NotImplementedError("write your pallas kernel here")


# speedup vs baseline: 7.3143x; 7.3143x over previous
import functools

import jax
import jax.numpy as jnp
from jax import lax
from jax.experimental import pallas as pl
from jax.experimental.pallas import tpu as pltpu
from jax.experimental.pallas import tpu_sc as plsc

N = 10000
E = 320000
D = 128
L = 5

NC = 2
NS = 16
NW = NC * NS

EB = 80
ENB = N // EB
EK = -(-ENB // NW)

AB = 125
ANB = (E // NW) // AB
ZR = 632
ZLAST = N - (NS - 1) * ZR


def _embed_kernel(embed_hbm, xidx_hbm, out_hbm, idx_v, rows_v):
    wid = lax.axis_index("s") * NC + lax.axis_index("c")

    @pl.loop(0, EK)
    def _(k):
        b = wid + k * NW

        @pl.when(b < ENB)
        def _():
            pltpu.sync_copy(xidx_hbm.at[b], idx_v)
            pltpu.sync_copy(embed_hbm.at[idx_v], rows_v)
            pltpu.sync_copy(rows_v, out_hbm.at[pl.ds(b * EB, EB)])


def _agg_kernel(h_hbm, src_hbm, dst_hbm, zeros_hbm, out_hbm,
                src_v, dst_v, rows_v, acc_shared):
    c = lax.axis_index("c")
    s = lax.axis_index("s")
    wid = s * NC + c
    @pl.when(s < NS - 1)
    def _():
        pltpu.sync_copy(zeros_hbm, acc_shared.at[pl.ds(s * ZR, ZR)])

    @pl.when(s == NS - 1)
    def _():
        pltpu.sync_copy(zeros_hbm.at[pl.ds(0, ZLAST)],
                        acc_shared.at[pl.ds((NS - 1) * ZR, ZLAST)])

    pltpu.sync_copy(src_hbm.at[wid], src_v)
    pltpu.sync_copy(dst_hbm.at[wid], dst_v)
    plsc.subcore_barrier()

    @pl.loop(0, ANB)
    def _(j):
        pltpu.sync_copy(h_hbm.at[src_v.at[j]], rows_v)
        pltpu.sync_copy(rows_v, acc_shared.at[dst_v.at[j]], add=True)

    plsc.subcore_barrier()

    @pl.when(s < NS - 1)
    def _():
        pltpu.sync_copy(acc_shared.at[pl.ds(s * ZR, ZR)],
                        out_hbm.at[c, pl.ds(s * ZR, ZR)])

    @pl.when(s == NS - 1)
    def _():
        pltpu.sync_copy(acc_shared.at[pl.ds((NS - 1) * ZR, ZLAST)],
                        out_hbm.at[c, pl.ds((NS - 1) * ZR, ZLAST)])


RB = 1000
GB = N // RB


def _lin_kernel(h_ref, w_ref, m_ref):
    m_ref[...] = jnp.dot(h_ref[...], w_ref[...],
                         preferred_element_type=jnp.float32)


def _gru_core(part_ref, h_ref, wih_ref, whh_ref, bih_ref, bhh_ref):
    g = part_ref[0] + part_ref[1]
    gi = lax.dot_general(g, wih_ref[...], (((1,), (1,)), ((), ())),
                         preferred_element_type=jnp.float32) + bih_ref[...]
    gh = lax.dot_general(h_ref[...], whh_ref[...], (((1,), (1,)), ((), ())),
                         preferred_element_type=jnp.float32) + bhh_ref[...]
    r = jax.nn.sigmoid(gi[:, 0:D] + gh[:, 0:D])
    z = jax.nn.sigmoid(gi[:, D:2 * D] + gh[:, D:2 * D])
    n = jnp.tanh(gi[:, 2 * D:] + r * gh[:, 2 * D:])
    return (1.0 - z) * n + z * h_ref[...]


def _gru_kernel(part_ref, h_ref, wnext_ref, wih_ref, whh_ref, bih_ref,
                bhh_ref, out_ref, m_ref):
    h_new = _gru_core(part_ref, h_ref, wih_ref, whh_ref, bih_ref, bhh_ref)
    out_ref[...] = h_new
    m_ref[...] = jnp.dot(h_new, wnext_ref[...],
                         preferred_element_type=jnp.float32)


def _gru_pool_kernel(part_ref, h_ref, wih_ref, whh_ref, bih_ref,
                     bhh_ref, gw_ref, gb_ref, hg_ref, num_acc, den_acc):
    i = pl.program_id(0)
    h_new = _gru_core(part_ref, h_ref, wih_ref, whh_ref, bih_ref, bhh_ref)
    gate = jax.nn.sigmoid(
        jnp.sum(h_new * gw_ref[...], axis=1, keepdims=True) + gb_ref[...])
    e = jnp.exp(gate)
    blk_num = jnp.sum(e * h_new, axis=0, keepdims=True)
    blk_den = jnp.sum(e)

    @pl.when(i == 0)
    def _():
        num_acc[...] = blk_num
        den_acc[0, 0] = blk_den

    @pl.when(i > 0)
    def _():
        num_acc[...] += blk_num
        den_acc[0, 0] += blk_den

    @pl.when(i == GB - 1)
    def _():
        hg_ref[...] = num_acc[...] / den_acc[0, 0]


def _sc_mesh():
    return plsc.VectorSubcoreMesh(core_axis_name="c", subcore_axis_name="s")


def _embed_call(embed, xidx):
    kern = pl.kernel(
        _embed_kernel,
        out_type=jax.ShapeDtypeStruct((N, D), jnp.float32),
        mesh=_sc_mesh(),
        scratch_types=[
            pltpu.VMEM((EB,), jnp.int32),
            pltpu.VMEM((EB, D), jnp.float32),
        ],
    )
    return kern(embed, xidx)


def _agg_call(h, src_r, dst_r, zeros):
    kern = pl.kernel(
        _agg_kernel,
        out_type=jax.ShapeDtypeStruct((NC, N, D), jnp.float32),
        mesh=_sc_mesh(),
        scratch_types=[
            pltpu.VMEM((ANB, AB), jnp.int32),
            pltpu.VMEM((ANB, AB), jnp.int32),
            pltpu.VMEM((AB, D), jnp.float32),
            pltpu.VMEM_SHARED((N, D), jnp.float32),
        ],
    )
    return kern(h, src_r, dst_r, zeros)


_GRU_WEIGHT_SPECS = [
    pl.BlockSpec((3 * D, D), lambda i: (0, 0)),
    pl.BlockSpec((3 * D, D), lambda i: (0, 0)),
    pl.BlockSpec((1, 3 * D), lambda i: (0, 0)),
    pl.BlockSpec((1, 3 * D), lambda i: (0, 0)),
]


def _lin_call(h, Wi):
    return pl.pallas_call(
        _lin_kernel,
        grid=(GB,),
        in_specs=[
            pl.BlockSpec((RB, D), lambda i: (i, 0)),
            pl.BlockSpec((D, D), lambda i: (0, 0)),
        ],
        out_specs=pl.BlockSpec((RB, D), lambda i: (i, 0)),
        out_shape=jax.ShapeDtypeStruct((N, D), jnp.float32),
    )(h, Wi)


def _gru_call(part, h, Wnext, W_ih, W_hh, bih, bhh):
    return pl.pallas_call(
        _gru_kernel,
        grid=(GB,),
        in_specs=[
            pl.BlockSpec((NC, RB, D), lambda i: (0, i, 0)),
            pl.BlockSpec((RB, D), lambda i: (i, 0)),
            pl.BlockSpec((D, D), lambda i: (0, 0)),
        ] + _GRU_WEIGHT_SPECS,
        out_specs=[
            pl.BlockSpec((RB, D), lambda i: (i, 0)),
            pl.BlockSpec((RB, D), lambda i: (i, 0)),
        ],
        out_shape=[
            jax.ShapeDtypeStruct((N, D), jnp.float32),
            jax.ShapeDtypeStruct((N, D), jnp.float32),
        ],
    )(part, h, Wnext, W_ih, W_hh, bih, bhh)


def _gru_pool_call(part, h, W_ih, W_hh, bih, bhh, gw, gb):
    return pl.pallas_call(
        _gru_pool_kernel,
        grid=(GB,),
        in_specs=[
            pl.BlockSpec((NC, RB, D), lambda i: (0, i, 0)),
            pl.BlockSpec((RB, D), lambda i: (i, 0)),
        ] + _GRU_WEIGHT_SPECS + [
            pl.BlockSpec((1, D), lambda i: (0, 0)),
            pl.BlockSpec((1, 1), lambda i: (0, 0)),
        ],
        out_specs=pl.BlockSpec((1, D), lambda i: (0, 0)),
        out_shape=jax.ShapeDtypeStruct((1, D), jnp.float32),
        scratch_shapes=[
            pltpu.VMEM((1, D), jnp.float32),
            pltpu.SMEM((1, 1), jnp.float32),
        ],
    )(part, h, W_ih, W_hh, bih, bhh, gw, gb)


def kernel(x, edge_index, embed, W, W_ih, W_hh, b_ih, b_hh, gate_w, gate_b):
    xidx = x[:, 0].reshape(ENB, EB)
    src_r = edge_index[0].reshape(NW, ANB, AB)
    dst_r = edge_index[1].reshape(NW, ANB, AB)
    zeros = jnp.zeros((ZR, D), jnp.float32)
    bih = b_ih.reshape(1, 3 * D)
    bhh = b_hh.reshape(1, 3 * D)
    gb = gate_b.reshape(1, 1)

    h = _embed_call(embed, xidx)
    m = _lin_call(h, W[0])
    for i in range(L):
        part = _agg_call(m, src_r, dst_r, zeros)
        if i < L - 1:
            h, m = _gru_call(part, h, W[i + 1], W_ih, W_hh, bih, bhh)
        else:
            hg = _gru_pool_call(part, h, W_ih, W_hh, bih, bhh, gate_w, gb)
    return hg


# speedup vs baseline: 10.9467x; 1.4966x over previous
import functools

import jax
import jax.numpy as jnp
from jax import lax
from jax.experimental import pallas as pl
from jax.experimental.pallas import tpu as pltpu
from jax.experimental.pallas import tpu_sc as plsc

N = 10000
E = 320000
D = 128
L = 5

NC = 2
NS = 16
NW = NC * NS

EB = 80
ENB = N // EB
EK = -(-ENB // NW)

AB = 125
ANB = (E // NW) // AB
HH = ANB // 2
ZR = 632
ZLAST = N - (NS - 1) * ZR


def _embed_kernel(embed_hbm, xidx_hbm, out_hbm, idx_v, rows_v):
    wid = lax.axis_index("s") * NC + lax.axis_index("c")

    @pl.loop(0, EK)
    def _(k):
        b = wid + k * NW

        @pl.when(b < ENB)
        def _():
            pltpu.sync_copy(xidx_hbm.at[b], idx_v)
            pltpu.sync_copy(embed_hbm.at[idx_v], rows_v)
            pltpu.sync_copy(rows_v, out_hbm.at[pl.ds(b * EB, EB)])


def _agg_kernel(h_hbm, src_hbm, dst_hbm, zeros_hbm, out_hbm,
                src_v, dst_v, rows0, rows1, gsem0, gsem1, ssem0, ssem1,
                acc_shared):
    c = lax.axis_index("c")
    s = lax.axis_index("s")
    wid = s * NC + c
    pltpu.sync_copy(src_hbm.at[wid, pl.ds(0, HH)], src_v)
    pltpu.sync_copy(dst_hbm.at[wid, pl.ds(0, HH)], dst_v)
    pltpu.async_copy(h_hbm.at[src_v.at[0]], rows0, gsem0)
    pltpu.async_copy(h_hbm.at[src_v.at[1]], rows1, gsem1)

    @pl.when(s < NS - 1)
    def _():
        pltpu.sync_copy(zeros_hbm, acc_shared.at[pl.ds(s * ZR, ZR)])

    @pl.when(s == NS - 1)
    def _():
        pltpu.sync_copy(zeros_hbm.at[pl.ds(0, ZLAST)],
                        acc_shared.at[pl.ds((NS - 1) * ZR, ZLAST)])

    plsc.subcore_barrier()

    for half in range(2):
        @pl.loop(0, HH, step=2)
        def _(j):
            @pl.when(j > 0)
            def _():
                pltpu.make_async_copy(rows1, acc_shared.at[dst_v.at[j - 1]],
                                      ssem1).wait()
                pltpu.async_copy(h_hbm.at[src_v.at[j + 1]], rows1, gsem1)

            pltpu.make_async_copy(h_hbm.at[src_v.at[j]], rows0, gsem0).wait()
            pltpu.async_copy(rows0, acc_shared.at[dst_v.at[j]], ssem0,
                             add=True)
            pltpu.make_async_copy(rows0, acc_shared.at[dst_v.at[j]],
                                  ssem0).wait()

            @pl.when(j + 2 < HH)
            def _():
                pltpu.async_copy(h_hbm.at[src_v.at[j + 2]], rows0, gsem0)

            pltpu.make_async_copy(h_hbm.at[src_v.at[j + 1]], rows1,
                                  gsem1).wait()
            pltpu.async_copy(rows1, acc_shared.at[dst_v.at[j + 1]], ssem1,
                             add=True)

        pltpu.make_async_copy(rows1, acc_shared.at[dst_v.at[HH - 1]],
                              ssem1).wait()
        if half == 0:
            pltpu.sync_copy(src_hbm.at[wid, pl.ds(HH, HH)], src_v)
            pltpu.sync_copy(dst_hbm.at[wid, pl.ds(HH, HH)], dst_v)
            pltpu.async_copy(h_hbm.at[src_v.at[0]], rows0, gsem0)
            pltpu.async_copy(h_hbm.at[src_v.at[1]], rows1, gsem1)

    plsc.subcore_barrier()

    @pl.when(s < NS - 1)
    def _():
        pltpu.sync_copy(acc_shared.at[pl.ds(s * ZR, ZR)],
                        out_hbm.at[c, pl.ds(s * ZR, ZR)])

    @pl.when(s == NS - 1)
    def _():
        pltpu.sync_copy(acc_shared.at[pl.ds((NS - 1) * ZR, ZLAST)],
                        out_hbm.at[c, pl.ds((NS - 1) * ZR, ZLAST)])


RB = 1000
GB = N // RB


def _lin_kernel(h_ref, w_ref, m_ref):
    m_ref[...] = jnp.dot(h_ref[...], w_ref[...],
                         preferred_element_type=jnp.float32)


def _gru_core(part_ref, h_ref, wih_ref, whh_ref, bih_ref, bhh_ref):
    g = part_ref[0] + part_ref[1]
    gi = lax.dot_general(g, wih_ref[...], (((1,), (1,)), ((), ())),
                         preferred_element_type=jnp.float32) + bih_ref[...]
    gh = lax.dot_general(h_ref[...], whh_ref[...], (((1,), (1,)), ((), ())),
                         preferred_element_type=jnp.float32) + bhh_ref[...]
    r = jax.nn.sigmoid(gi[:, 0:D] + gh[:, 0:D])
    z = jax.nn.sigmoid(gi[:, D:2 * D] + gh[:, D:2 * D])
    n = jnp.tanh(gi[:, 2 * D:] + r * gh[:, 2 * D:])
    return (1.0 - z) * n + z * h_ref[...]


def _gru_kernel(part_ref, h_ref, wnext_ref, wih_ref, whh_ref, bih_ref,
                bhh_ref, out_ref, m_ref):
    h_new = _gru_core(part_ref, h_ref, wih_ref, whh_ref, bih_ref, bhh_ref)
    out_ref[...] = h_new
    m_ref[...] = jnp.dot(h_new, wnext_ref[...],
                         preferred_element_type=jnp.float32)


def _gru_pool_kernel(part_ref, h_ref, wih_ref, whh_ref, bih_ref,
                     bhh_ref, gw_ref, gb_ref, hg_ref, num_acc, den_acc):
    i = pl.program_id(0)
    h_new = _gru_core(part_ref, h_ref, wih_ref, whh_ref, bih_ref, bhh_ref)
    gate = jax.nn.sigmoid(
        jnp.sum(h_new * gw_ref[...], axis=1, keepdims=True) + gb_ref[...])
    e = jnp.exp(gate)
    blk_num = jnp.sum(e * h_new, axis=0, keepdims=True)
    blk_den = jnp.sum(e)

    @pl.when(i == 0)
    def _():
        num_acc[...] = blk_num
        den_acc[0, 0] = blk_den

    @pl.when(i > 0)
    def _():
        num_acc[...] += blk_num
        den_acc[0, 0] += blk_den

    @pl.when(i == GB - 1)
    def _():
        hg_ref[...] = num_acc[...] / den_acc[0, 0]


def _sc_mesh():
    return plsc.VectorSubcoreMesh(core_axis_name="c", subcore_axis_name="s")


def _embed_call(embed, xidx):
    kern = pl.kernel(
        _embed_kernel,
        out_type=jax.ShapeDtypeStruct((N, D), jnp.float32),
        mesh=_sc_mesh(),
        scratch_types=[
            pltpu.VMEM((EB,), jnp.int32),
            pltpu.VMEM((EB, D), jnp.float32),
        ],
    )
    return kern(embed, xidx)


def _agg_call(h, src_r, dst_r, zeros):
    kern = pl.kernel(
        _agg_kernel,
        out_type=jax.ShapeDtypeStruct((NC, N, D), jnp.float32),
        mesh=_sc_mesh(),
        scratch_types=[
            pltpu.VMEM((HH, AB), jnp.int32),
            pltpu.VMEM((HH, AB), jnp.int32),
            pltpu.VMEM((AB, D), jnp.float32),
            pltpu.VMEM((AB, D), jnp.float32),
            pltpu.SemaphoreType.DMA,
            pltpu.SemaphoreType.DMA,
            pltpu.SemaphoreType.DMA,
            pltpu.SemaphoreType.DMA,
            pltpu.VMEM_SHARED((N, D), jnp.float32),
        ],
    )
    return kern(h, src_r, dst_r, zeros)


_GRU_WEIGHT_SPECS = [
    pl.BlockSpec((3 * D, D), lambda i: (0, 0)),
    pl.BlockSpec((3 * D, D), lambda i: (0, 0)),
    pl.BlockSpec((1, 3 * D), lambda i: (0, 0)),
    pl.BlockSpec((1, 3 * D), lambda i: (0, 0)),
]


def _lin_call(h, Wi):
    return pl.pallas_call(
        _lin_kernel,
        grid=(GB,),
        in_specs=[
            pl.BlockSpec((RB, D), lambda i: (i, 0)),
            pl.BlockSpec((D, D), lambda i: (0, 0)),
        ],
        out_specs=pl.BlockSpec((RB, D), lambda i: (i, 0)),
        out_shape=jax.ShapeDtypeStruct((N, D), jnp.float32),
    )(h, Wi)


def _gru_call(part, h, Wnext, W_ih, W_hh, bih, bhh):
    return pl.pallas_call(
        _gru_kernel,
        grid=(GB,),
        in_specs=[
            pl.BlockSpec((NC, RB, D), lambda i: (0, i, 0)),
            pl.BlockSpec((RB, D), lambda i: (i, 0)),
            pl.BlockSpec((D, D), lambda i: (0, 0)),
        ] + _GRU_WEIGHT_SPECS,
        out_specs=[
            pl.BlockSpec((RB, D), lambda i: (i, 0)),
            pl.BlockSpec((RB, D), lambda i: (i, 0)),
        ],
        out_shape=[
            jax.ShapeDtypeStruct((N, D), jnp.float32),
            jax.ShapeDtypeStruct((N, D), jnp.float32),
        ],
    )(part, h, Wnext, W_ih, W_hh, bih, bhh)


def _gru_pool_call(part, h, W_ih, W_hh, bih, bhh, gw, gb):
    return pl.pallas_call(
        _gru_pool_kernel,
        grid=(GB,),
        in_specs=[
            pl.BlockSpec((NC, RB, D), lambda i: (0, i, 0)),
            pl.BlockSpec((RB, D), lambda i: (i, 0)),
        ] + _GRU_WEIGHT_SPECS + [
            pl.BlockSpec((1, D), lambda i: (0, 0)),
            pl.BlockSpec((1, 1), lambda i: (0, 0)),
        ],
        out_specs=pl.BlockSpec((1, D), lambda i: (0, 0)),
        out_shape=jax.ShapeDtypeStruct((1, D), jnp.float32),
        scratch_shapes=[
            pltpu.VMEM((1, D), jnp.float32),
            pltpu.SMEM((1, 1), jnp.float32),
        ],
    )(part, h, W_ih, W_hh, bih, bhh, gw, gb)


def kernel(x, edge_index, embed, W, W_ih, W_hh, b_ih, b_hh, gate_w, gate_b):
    xidx = x[:, 0].reshape(ENB, EB)
    src_r = edge_index[0].reshape(NW, ANB, AB)
    dst_r = edge_index[1].reshape(NW, ANB, AB)
    zeros = jnp.zeros((ZR, D), jnp.float32)
    bih = b_ih.reshape(1, 3 * D)
    bhh = b_hh.reshape(1, 3 * D)
    gb = gate_b.reshape(1, 1)

    h = _embed_call(embed, xidx)
    m = _lin_call(h, W[0])
    for i in range(L):
        part = _agg_call(m, src_r, dst_r, zeros)
        if i < L - 1:
            h, m = _gru_call(part, h, W[i + 1], W_ih, W_hh, bih, bhh)
        else:
            hg = _gru_pool_call(part, h, W_ih, W_hh, bih, bhh, gate_w, gb)
    return hg


# speedup vs baseline: 11.0895x; 1.0130x over previous
import functools

import jax
import jax.numpy as jnp
from jax import lax
from jax.experimental import pallas as pl
from jax.experimental.pallas import tpu as pltpu
from jax.experimental.pallas import tpu_sc as plsc

N = 10000
E = 320000
D = 128
L = 5

NC = 2
NS = 16
NW = NC * NS

EB = 80
ENB = N // EB
EK = -(-ENB // NW)

AB = 125
ANB = (E // NW) // AB
HH = ANB // 2
ZR = 632
ZLAST = N - (NS - 1) * ZR


def _embed_kernel(embed_hbm, xidx_hbm, out_hbm, idx_v, rows_v):
    wid = lax.axis_index("s") * NC + lax.axis_index("c")

    @pl.loop(0, EK)
    def _(k):
        b = wid + k * NW

        @pl.when(b < ENB)
        def _():
            pltpu.sync_copy(xidx_hbm.at[b], idx_v)
            pltpu.sync_copy(embed_hbm.at[idx_v], rows_v)
            pltpu.sync_copy(rows_v, out_hbm.at[pl.ds(b * EB, EB)])


def _agg_kernel(h_hbm, src_hbm, dst_hbm, zeros_hbm, out_hbm,
                src_v, dst_v, rows0, rows1, gsem0, gsem1, ssem0, ssem1,
                acc_shared):
    c = lax.axis_index("c")
    s = lax.axis_index("s")
    wid = s * NC + c
    pltpu.sync_copy(src_hbm.at[wid, pl.ds(0, HH)], src_v)
    pltpu.sync_copy(dst_hbm.at[wid, pl.ds(0, HH)], dst_v)
    pltpu.async_copy(h_hbm.at[src_v.at[0]], rows0, gsem0)
    pltpu.async_copy(h_hbm.at[src_v.at[1]], rows1, gsem1)

    @pl.when(s < NS - 1)
    def _():
        pltpu.sync_copy(zeros_hbm, acc_shared.at[pl.ds(s * ZR, ZR)])

    @pl.when(s == NS - 1)
    def _():
        pltpu.sync_copy(zeros_hbm.at[pl.ds(0, ZLAST)],
                        acc_shared.at[pl.ds((NS - 1) * ZR, ZLAST)])

    plsc.subcore_barrier()

    for half in range(2):
        @pl.loop(0, HH, step=2)
        def _(j):
            @pl.when(j > 0)
            def _():
                pltpu.make_async_copy(rows1, acc_shared.at[dst_v.at[j - 1]],
                                      ssem1).wait()
                pltpu.async_copy(h_hbm.at[src_v.at[j + 1]], rows1, gsem1)

            pltpu.make_async_copy(h_hbm.at[src_v.at[j]], rows0, gsem0).wait()
            pltpu.async_copy(rows0, acc_shared.at[dst_v.at[j]], ssem0,
                             add=True)
            pltpu.make_async_copy(rows0, acc_shared.at[dst_v.at[j]],
                                  ssem0).wait()

            @pl.when(j + 2 < HH)
            def _():
                pltpu.async_copy(h_hbm.at[src_v.at[j + 2]], rows0, gsem0)

            pltpu.make_async_copy(h_hbm.at[src_v.at[j + 1]], rows1,
                                  gsem1).wait()
            pltpu.async_copy(rows1, acc_shared.at[dst_v.at[j + 1]], ssem1,
                             add=True)

        pltpu.make_async_copy(rows1, acc_shared.at[dst_v.at[HH - 1]],
                              ssem1).wait()
        if half == 0:
            pltpu.sync_copy(src_hbm.at[wid, pl.ds(HH, HH)], src_v)
            pltpu.sync_copy(dst_hbm.at[wid, pl.ds(HH, HH)], dst_v)
            pltpu.async_copy(h_hbm.at[src_v.at[0]], rows0, gsem0)
            pltpu.async_copy(h_hbm.at[src_v.at[1]], rows1, gsem1)

    plsc.subcore_barrier()

    @pl.when(s < NS - 1)
    def _():
        pltpu.sync_copy(acc_shared.at[pl.ds(s * ZR, ZR)],
                        out_hbm.at[c, pl.ds(s * ZR, ZR)])

    @pl.when(s == NS - 1)
    def _():
        pltpu.sync_copy(acc_shared.at[pl.ds((NS - 1) * ZR, ZLAST)],
                        out_hbm.at[c, pl.ds((NS - 1) * ZR, ZLAST)])


RB = 2000
GB = N // RB


def _lin_kernel(h_ref, w_ref, m_ref):
    m_ref[...] = jnp.dot(h_ref[...], w_ref[...],
                         preferred_element_type=jnp.float32)


def _gru_core(part_ref, h_ref, wih_ref, whh_ref, bih_ref, bhh_ref):
    g = part_ref[0] + part_ref[1]
    gi = lax.dot_general(g, wih_ref[...], (((1,), (1,)), ((), ())),
                         preferred_element_type=jnp.float32) + bih_ref[...]
    gh = lax.dot_general(h_ref[...], whh_ref[...], (((1,), (1,)), ((), ())),
                         preferred_element_type=jnp.float32) + bhh_ref[...]
    r = jax.nn.sigmoid(gi[:, 0:D] + gh[:, 0:D])
    z = jax.nn.sigmoid(gi[:, D:2 * D] + gh[:, D:2 * D])
    n = jnp.tanh(gi[:, 2 * D:] + r * gh[:, 2 * D:])
    return (1.0 - z) * n + z * h_ref[...]


def _gru_kernel(part_ref, h_ref, wnext_ref, wih_ref, whh_ref, bih_ref,
                bhh_ref, out_ref, m_ref):
    h_new = _gru_core(part_ref, h_ref, wih_ref, whh_ref, bih_ref, bhh_ref)
    out_ref[...] = h_new
    m_ref[...] = jnp.dot(h_new, wnext_ref[...],
                         preferred_element_type=jnp.float32)


def _gru_pool_kernel(part_ref, h_ref, wih_ref, whh_ref, bih_ref,
                     bhh_ref, gw_ref, gb_ref, hg_ref, num_acc, den_acc):
    i = pl.program_id(0)
    h_new = _gru_core(part_ref, h_ref, wih_ref, whh_ref, bih_ref, bhh_ref)
    gate = jax.nn.sigmoid(
        jnp.sum(h_new * gw_ref[...], axis=1, keepdims=True) + gb_ref[...])
    e = jnp.exp(gate)
    blk_num = jnp.sum(e * h_new, axis=0, keepdims=True)
    blk_den = jnp.sum(e)

    @pl.when(i == 0)
    def _():
        num_acc[...] = blk_num
        den_acc[0, 0] = blk_den

    @pl.when(i > 0)
    def _():
        num_acc[...] += blk_num
        den_acc[0, 0] += blk_den

    @pl.when(i == GB - 1)
    def _():
        hg_ref[...] = num_acc[...] / den_acc[0, 0]


def _sc_mesh():
    return plsc.VectorSubcoreMesh(core_axis_name="c", subcore_axis_name="s")


def _embed_call(embed, xidx):
    kern = pl.kernel(
        _embed_kernel,
        out_type=jax.ShapeDtypeStruct((N, D), jnp.float32),
        mesh=_sc_mesh(),
        scratch_types=[
            pltpu.VMEM((EB,), jnp.int32),
            pltpu.VMEM((EB, D), jnp.float32),
        ],
    )
    return kern(embed, xidx)


def _agg_call(h, src_r, dst_r, zeros):
    kern = pl.kernel(
        _agg_kernel,
        out_type=jax.ShapeDtypeStruct((NC, N, D), jnp.float32),
        mesh=_sc_mesh(),
        scratch_types=[
            pltpu.VMEM((HH, AB), jnp.int32),
            pltpu.VMEM((HH, AB), jnp.int32),
            pltpu.VMEM((AB, D), jnp.float32),
            pltpu.VMEM((AB, D), jnp.float32),
            pltpu.SemaphoreType.DMA,
            pltpu.SemaphoreType.DMA,
            pltpu.SemaphoreType.DMA,
            pltpu.SemaphoreType.DMA,
            pltpu.VMEM_SHARED((N, D), jnp.float32),
        ],
    )
    return kern(h, src_r, dst_r, zeros)


_GRU_WEIGHT_SPECS = [
    pl.BlockSpec((3 * D, D), lambda i: (0, 0)),
    pl.BlockSpec((3 * D, D), lambda i: (0, 0)),
    pl.BlockSpec((1, 3 * D), lambda i: (0, 0)),
    pl.BlockSpec((1, 3 * D), lambda i: (0, 0)),
]


def _lin_call(h, Wi):
    return pl.pallas_call(
        _lin_kernel,
        grid=(GB,),
        in_specs=[
            pl.BlockSpec((RB, D), lambda i: (i, 0)),
            pl.BlockSpec((D, D), lambda i: (0, 0)),
        ],
        out_specs=pl.BlockSpec((RB, D), lambda i: (i, 0)),
        out_shape=jax.ShapeDtypeStruct((N, D), jnp.float32),
    )(h, Wi)


def _gru_call(part, h, Wnext, W_ih, W_hh, bih, bhh):
    return pl.pallas_call(
        _gru_kernel,
        grid=(GB,),
        in_specs=[
            pl.BlockSpec((NC, RB, D), lambda i: (0, i, 0)),
            pl.BlockSpec((RB, D), lambda i: (i, 0)),
            pl.BlockSpec((D, D), lambda i: (0, 0)),
        ] + _GRU_WEIGHT_SPECS,
        out_specs=[
            pl.BlockSpec((RB, D), lambda i: (i, 0)),
            pl.BlockSpec((RB, D), lambda i: (i, 0)),
        ],
        out_shape=[
            jax.ShapeDtypeStruct((N, D), jnp.float32),
            jax.ShapeDtypeStruct((N, D), jnp.float32),
        ],
    )(part, h, Wnext, W_ih, W_hh, bih, bhh)


def _gru_pool_call(part, h, W_ih, W_hh, bih, bhh, gw, gb):
    return pl.pallas_call(
        _gru_pool_kernel,
        grid=(GB,),
        in_specs=[
            pl.BlockSpec((NC, RB, D), lambda i: (0, i, 0)),
            pl.BlockSpec((RB, D), lambda i: (i, 0)),
        ] + _GRU_WEIGHT_SPECS + [
            pl.BlockSpec((1, D), lambda i: (0, 0)),
            pl.BlockSpec((1, 1), lambda i: (0, 0)),
        ],
        out_specs=pl.BlockSpec((1, D), lambda i: (0, 0)),
        out_shape=jax.ShapeDtypeStruct((1, D), jnp.float32),
        scratch_shapes=[
            pltpu.VMEM((1, D), jnp.float32),
            pltpu.SMEM((1, 1), jnp.float32),
        ],
    )(part, h, W_ih, W_hh, bih, bhh, gw, gb)


def kernel(x, edge_index, embed, W, W_ih, W_hh, b_ih, b_hh, gate_w, gate_b):
    xidx = x[:, 0].reshape(ENB, EB)
    src_r = edge_index[0].reshape(NW, ANB, AB)
    dst_r = edge_index[1].reshape(NW, ANB, AB)
    zeros = jnp.zeros((ZR, D), jnp.float32)
    bih = b_ih.reshape(1, 3 * D)
    bhh = b_hh.reshape(1, 3 * D)
    gb = gate_b.reshape(1, 1)

    h = _embed_call(embed, xidx)
    m = _lin_call(h, W[0])
    for i in range(L):
        part = _agg_call(m, src_r, dst_r, zeros)
        if i < L - 1:
            h, m = _gru_call(part, h, W[i + 1], W_ih, W_hh, bih, bhh)
        else:
            hg = _gru_pool_call(part, h, W_ih, W_hh, bih, bhh, gate_w, gb)
    return hg


# speedup vs baseline: 11.1443x; 1.0049x over previous
import functools

import jax
import jax.numpy as jnp
from jax import lax
from jax.experimental import pallas as pl
from jax.experimental.pallas import tpu as pltpu
from jax.experimental.pallas import tpu_sc as plsc

N = 10000
E = 320000
D = 128
L = 5

NC = 2
NS = 16
NW = NC * NS

EB = 80
EK = 4
NPAD = NW * EK * EB

AB = 125
ANB = (E // NW) // AB
HH = ANB // 2
ZR = 632
ZLAST = N - (NS - 1) * ZR


def _embed_kernel(embed_hbm, xidx_hbm, out_hbm, idx_v, rows0, rows1,
                  gsem0, gsem1, wsem0, wsem1):
    wid = lax.axis_index("s") * NC + lax.axis_index("c")
    base = wid * EK * EB
    pltpu.sync_copy(xidx_hbm.at[wid], idx_v)
    pltpu.async_copy(embed_hbm.at[idx_v.at[0]], rows0, gsem0)

    def _wr(rows, k, wsem):
        return pltpu.make_async_copy(
            rows, out_hbm.at[pl.ds(base + k * EB, EB)], wsem)

    @pl.when(wid == NW - 1)
    def _():
        pltpu.make_async_copy(embed_hbm.at[idx_v.at[0]], rows0, gsem0).wait()
        pltpu.sync_copy(rows0, out_hbm.at[pl.ds(base, EB)])

    @pl.when(wid < NW - 1)
    def _():
        pltpu.async_copy(embed_hbm.at[idx_v.at[1]], rows1, gsem1)
        pltpu.make_async_copy(embed_hbm.at[idx_v.at[0]], rows0, gsem0).wait()
        _wr(rows0, 0, wsem0).start()
        pltpu.make_async_copy(embed_hbm.at[idx_v.at[1]], rows1, gsem1).wait()
        _wr(rows1, 1, wsem1).start()
        _wr(rows0, 0, wsem0).wait()
        pltpu.async_copy(embed_hbm.at[idx_v.at[2]], rows0, gsem0)
        _wr(rows1, 1, wsem1).wait()
        pltpu.async_copy(embed_hbm.at[idx_v.at[3]], rows1, gsem1)
        pltpu.make_async_copy(embed_hbm.at[idx_v.at[2]], rows0, gsem0).wait()
        _wr(rows0, 2, wsem0).start()
        pltpu.make_async_copy(embed_hbm.at[idx_v.at[3]], rows1, gsem1).wait()
        _wr(rows1, 3, wsem1).start()
        _wr(rows0, 2, wsem0).wait()
        _wr(rows1, 3, wsem1).wait()


def _agg_kernel(h_hbm, src_hbm, dst_hbm, zeros_hbm, out_hbm,
                src_v, dst_v, rows0, rows1, gsem0, gsem1, ssem0, ssem1,
                acc_shared):
    c = lax.axis_index("c")
    s = lax.axis_index("s")
    wid = s * NC + c
    pltpu.sync_copy(src_hbm.at[wid, pl.ds(0, HH)], src_v)
    pltpu.sync_copy(dst_hbm.at[wid, pl.ds(0, HH)], dst_v)
    pltpu.async_copy(h_hbm.at[src_v.at[0]], rows0, gsem0)
    pltpu.async_copy(h_hbm.at[src_v.at[1]], rows1, gsem1)

    @pl.when(s < NS - 1)
    def _():
        pltpu.sync_copy(zeros_hbm, acc_shared.at[pl.ds(s * ZR, ZR)])

    @pl.when(s == NS - 1)
    def _():
        pltpu.sync_copy(zeros_hbm.at[pl.ds(0, ZLAST)],
                        acc_shared.at[pl.ds((NS - 1) * ZR, ZLAST)])

    plsc.subcore_barrier()

    for half in range(2):
        @pl.loop(0, HH, step=2)
        def _(j):
            @pl.when(j > 0)
            def _():
                pltpu.make_async_copy(rows1, acc_shared.at[dst_v.at[j - 1]],
                                      ssem1).wait()
                pltpu.async_copy(h_hbm.at[src_v.at[j + 1]], rows1, gsem1)

            pltpu.make_async_copy(h_hbm.at[src_v.at[j]], rows0, gsem0).wait()
            pltpu.async_copy(rows0, acc_shared.at[dst_v.at[j]], ssem0,
                             add=True)
            pltpu.make_async_copy(rows0, acc_shared.at[dst_v.at[j]],
                                  ssem0).wait()

            @pl.when(j + 2 < HH)
            def _():
                pltpu.async_copy(h_hbm.at[src_v.at[j + 2]], rows0, gsem0)

            pltpu.make_async_copy(h_hbm.at[src_v.at[j + 1]], rows1,
                                  gsem1).wait()
            pltpu.async_copy(rows1, acc_shared.at[dst_v.at[j + 1]], ssem1,
                             add=True)

        pltpu.make_async_copy(rows1, acc_shared.at[dst_v.at[HH - 1]],
                              ssem1).wait()
        if half == 0:
            pltpu.sync_copy(src_hbm.at[wid, pl.ds(HH, HH)], src_v)
            pltpu.sync_copy(dst_hbm.at[wid, pl.ds(HH, HH)], dst_v)
            pltpu.async_copy(h_hbm.at[src_v.at[0]], rows0, gsem0)
            pltpu.async_copy(h_hbm.at[src_v.at[1]], rows1, gsem1)

    plsc.subcore_barrier()

    @pl.when(s < NS - 1)
    def _():
        pltpu.sync_copy(acc_shared.at[pl.ds(s * ZR, ZR)],
                        out_hbm.at[c, pl.ds(s * ZR, ZR)])

    @pl.when(s == NS - 1)
    def _():
        pltpu.sync_copy(acc_shared.at[pl.ds((NS - 1) * ZR, ZLAST)],
                        out_hbm.at[c, pl.ds((NS - 1) * ZR, ZLAST)])


RB = 2000
GB = N // RB


def _lin_kernel(h_ref, w_ref, m_ref):
    m_ref[...] = jnp.dot(h_ref[...], w_ref[...],
                         preferred_element_type=jnp.float32)


def _gru_core(part_ref, h_ref, wih_ref, whh_ref, bih_ref, bhh_ref):
    g = part_ref[0] + part_ref[1]
    gi = lax.dot_general(g, wih_ref[...], (((1,), (1,)), ((), ())),
                         preferred_element_type=jnp.float32) + bih_ref[...]
    gh = lax.dot_general(h_ref[...], whh_ref[...], (((1,), (1,)), ((), ())),
                         preferred_element_type=jnp.float32) + bhh_ref[...]
    r = jax.nn.sigmoid(gi[:, 0:D] + gh[:, 0:D])
    z = jax.nn.sigmoid(gi[:, D:2 * D] + gh[:, D:2 * D])
    n = jnp.tanh(gi[:, 2 * D:] + r * gh[:, 2 * D:])
    return (1.0 - z) * n + z * h_ref[...]


def _gru_kernel(part_ref, h_ref, wnext_ref, wih_ref, whh_ref, bih_ref,
                bhh_ref, out_ref, m_ref):
    h_new = _gru_core(part_ref, h_ref, wih_ref, whh_ref, bih_ref, bhh_ref)
    out_ref[...] = h_new
    m_ref[...] = jnp.dot(h_new, wnext_ref[...],
                         preferred_element_type=jnp.float32)


def _gru_pool_kernel(part_ref, h_ref, wih_ref, whh_ref, bih_ref,
                     bhh_ref, gw_ref, gb_ref, hg_ref, num_acc, den_acc):
    i = pl.program_id(0)
    h_new = _gru_core(part_ref, h_ref, wih_ref, whh_ref, bih_ref, bhh_ref)
    gate = jax.nn.sigmoid(
        jnp.sum(h_new * gw_ref[...], axis=1, keepdims=True) + gb_ref[...])
    e = jnp.exp(gate)
    blk_num = jnp.sum(e * h_new, axis=0, keepdims=True)
    blk_den = jnp.sum(e)

    @pl.when(i == 0)
    def _():
        num_acc[...] = blk_num
        den_acc[0, 0] = blk_den

    @pl.when(i > 0)
    def _():
        num_acc[...] += blk_num
        den_acc[0, 0] += blk_den

    @pl.when(i == GB - 1)
    def _():
        hg_ref[...] = num_acc[...] / den_acc[0, 0]


def _sc_mesh():
    return plsc.VectorSubcoreMesh(core_axis_name="c", subcore_axis_name="s")


def _embed_call(embed, xidx):
    kern = pl.kernel(
        _embed_kernel,
        out_type=jax.ShapeDtypeStruct((N, D), jnp.float32),
        mesh=_sc_mesh(),
        scratch_types=[
            pltpu.VMEM((EK, EB), jnp.int32),
            pltpu.VMEM((EB, D), jnp.float32),
            pltpu.VMEM((EB, D), jnp.float32),
            pltpu.SemaphoreType.DMA,
            pltpu.SemaphoreType.DMA,
            pltpu.SemaphoreType.DMA,
            pltpu.SemaphoreType.DMA,
        ],
    )
    return kern(embed, xidx)


def _agg_call(h, src_r, dst_r, zeros):
    kern = pl.kernel(
        _agg_kernel,
        out_type=jax.ShapeDtypeStruct((NC, N, D), jnp.float32),
        mesh=_sc_mesh(),
        scratch_types=[
            pltpu.VMEM((HH, AB), jnp.int32),
            pltpu.VMEM((HH, AB), jnp.int32),
            pltpu.VMEM((AB, D), jnp.float32),
            pltpu.VMEM((AB, D), jnp.float32),
            pltpu.SemaphoreType.DMA,
            pltpu.SemaphoreType.DMA,
            pltpu.SemaphoreType.DMA,
            pltpu.SemaphoreType.DMA,
            pltpu.VMEM_SHARED((N, D), jnp.float32),
        ],
    )
    return kern(h, src_r, dst_r, zeros)


_GRU_WEIGHT_SPECS = [
    pl.BlockSpec((3 * D, D), lambda i: (0, 0)),
    pl.BlockSpec((3 * D, D), lambda i: (0, 0)),
    pl.BlockSpec((1, 3 * D), lambda i: (0, 0)),
    pl.BlockSpec((1, 3 * D), lambda i: (0, 0)),
]


def _lin_call(h, Wi):
    return pl.pallas_call(
        _lin_kernel,
        grid=(GB,),
        in_specs=[
            pl.BlockSpec((RB, D), lambda i: (i, 0)),
            pl.BlockSpec((D, D), lambda i: (0, 0)),
        ],
        out_specs=pl.BlockSpec((RB, D), lambda i: (i, 0)),
        out_shape=jax.ShapeDtypeStruct((N, D), jnp.float32),
    )(h, Wi)


def _gru_call(part, h, Wnext, W_ih, W_hh, bih, bhh):
    return pl.pallas_call(
        _gru_kernel,
        grid=(GB,),
        in_specs=[
            pl.BlockSpec((NC, RB, D), lambda i: (0, i, 0)),
            pl.BlockSpec((RB, D), lambda i: (i, 0)),
            pl.BlockSpec((D, D), lambda i: (0, 0)),
        ] + _GRU_WEIGHT_SPECS,
        out_specs=[
            pl.BlockSpec((RB, D), lambda i: (i, 0)),
            pl.BlockSpec((RB, D), lambda i: (i, 0)),
        ],
        out_shape=[
            jax.ShapeDtypeStruct((N, D), jnp.float32),
            jax.ShapeDtypeStruct((N, D), jnp.float32),
        ],
    )(part, h, Wnext, W_ih, W_hh, bih, bhh)


def _gru_pool_call(part, h, W_ih, W_hh, bih, bhh, gw, gb):
    return pl.pallas_call(
        _gru_pool_kernel,
        grid=(GB,),
        in_specs=[
            pl.BlockSpec((NC, RB, D), lambda i: (0, i, 0)),
            pl.BlockSpec((RB, D), lambda i: (i, 0)),
        ] + _GRU_WEIGHT_SPECS + [
            pl.BlockSpec((1, D), lambda i: (0, 0)),
            pl.BlockSpec((1, 1), lambda i: (0, 0)),
        ],
        out_specs=pl.BlockSpec((1, D), lambda i: (0, 0)),
        out_shape=jax.ShapeDtypeStruct((1, D), jnp.float32),
        scratch_shapes=[
            pltpu.VMEM((1, D), jnp.float32),
            pltpu.SMEM((1, 1), jnp.float32),
        ],
    )(part, h, W_ih, W_hh, bih, bhh, gw, gb)


def kernel(x, edge_index, embed, W, W_ih, W_hh, b_ih, b_hh, gate_w, gate_b):
    xidx = jnp.concatenate(
        [x[:, 0], jnp.zeros((NPAD - N,), jnp.int32)]).reshape(NW, EK, EB)
    src_r = edge_index[0].reshape(NW, ANB, AB)
    dst_r = edge_index[1].reshape(NW, ANB, AB)
    zeros = jnp.zeros((ZR, D), jnp.float32)
    bih = b_ih.reshape(1, 3 * D)
    bhh = b_hh.reshape(1, 3 * D)
    gb = gate_b.reshape(1, 1)

    h = _embed_call(embed, xidx)
    m = _lin_call(h, W[0])
    for i in range(L):
        part = _agg_call(m, src_r, dst_r, zeros)
        if i < L - 1:
            h, m = _gru_call(part, h, W[i + 1], W_ih, W_hh, bih, bhh)
        else:
            hg = _gru_pool_call(part, h, W_ih, W_hh, bih, bhh, gate_w, gb)
    return hg


# speedup vs baseline: 11.2326x; 1.0079x over previous
import functools

import jax
import jax.numpy as jnp
from jax import lax
from jax.experimental import pallas as pl
from jax.experimental.pallas import tpu as pltpu
from jax.experimental.pallas import tpu_sc as plsc

N = 10000
E = 320000
D = 128
L = 5

NC = 2
NS = 16
NW = NC * NS

EB = 80
EK = 4
NPAD = NW * EK * EB

AB = 125
ANB = (E // NW) // AB
HH = ANB // 2
ZR = 632
ZLAST = N - (NS - 1) * ZR


def _embed_kernel(embed_hbm, xidx_hbm, out_hbm, idx_v, rows0, rows1,
                  gsem0, gsem1, wsem0, wsem1):
    wid = lax.axis_index("s") * NC + lax.axis_index("c")
    base = wid * EK * EB
    pltpu.sync_copy(xidx_hbm.at[wid], idx_v)
    pltpu.async_copy(embed_hbm.at[idx_v.at[0]], rows0, gsem0)

    def _wr(rows, k, wsem):
        return pltpu.make_async_copy(
            rows, out_hbm.at[pl.ds(base + k * EB, EB)], wsem)

    @pl.when(wid == NW - 1)
    def _():
        pltpu.make_async_copy(embed_hbm.at[idx_v.at[0]], rows0, gsem0).wait()
        pltpu.sync_copy(rows0, out_hbm.at[pl.ds(base, EB)])

    @pl.when(wid < NW - 1)
    def _():
        pltpu.async_copy(embed_hbm.at[idx_v.at[1]], rows1, gsem1)
        pltpu.make_async_copy(embed_hbm.at[idx_v.at[0]], rows0, gsem0).wait()
        _wr(rows0, 0, wsem0).start()
        pltpu.make_async_copy(embed_hbm.at[idx_v.at[1]], rows1, gsem1).wait()
        _wr(rows1, 1, wsem1).start()
        _wr(rows0, 0, wsem0).wait()
        pltpu.async_copy(embed_hbm.at[idx_v.at[2]], rows0, gsem0)
        _wr(rows1, 1, wsem1).wait()
        pltpu.async_copy(embed_hbm.at[idx_v.at[3]], rows1, gsem1)
        pltpu.make_async_copy(embed_hbm.at[idx_v.at[2]], rows0, gsem0).wait()
        _wr(rows0, 2, wsem0).start()
        pltpu.make_async_copy(embed_hbm.at[idx_v.at[3]], rows1, gsem1).wait()
        _wr(rows1, 3, wsem1).start()
        _wr(rows0, 2, wsem0).wait()
        _wr(rows1, 3, wsem1).wait()


def _agg_kernel(h_hbm, src_hbm, dst_hbm, zeros_hbm, out_hbm,
                src_v, dst_v, rows0, rows1, gsem0, gsem1, ssem0, ssem1,
                acc_shared):
    c = lax.axis_index("c")
    s = lax.axis_index("s")
    wid = s * NC + c
    pltpu.sync_copy(src_hbm.at[wid, pl.ds(0, HH)], src_v)
    pltpu.sync_copy(dst_hbm.at[wid, pl.ds(0, HH)], dst_v)
    pltpu.async_copy(h_hbm.at[src_v.at[0]], rows0, gsem0)
    pltpu.async_copy(h_hbm.at[src_v.at[1]], rows1, gsem1)

    @pl.when(s < NS - 1)
    def _():
        pltpu.sync_copy(zeros_hbm.at[pl.ds(s * ZR, ZR)],
                        acc_shared.at[pl.ds(s * ZR, ZR)])

    @pl.when(s == NS - 1)
    def _():
        pltpu.sync_copy(zeros_hbm.at[pl.ds((NS - 1) * ZR, ZLAST)],
                        acc_shared.at[pl.ds((NS - 1) * ZR, ZLAST)])

    plsc.subcore_barrier()

    for half in range(2):
        @pl.loop(0, HH, step=2)
        def _(j):
            @pl.when(j > 0)
            def _():
                pltpu.make_async_copy(rows1, acc_shared.at[dst_v.at[j - 1]],
                                      ssem1).wait()
                pltpu.async_copy(h_hbm.at[src_v.at[j + 1]], rows1, gsem1)

            pltpu.make_async_copy(h_hbm.at[src_v.at[j]], rows0, gsem0).wait()
            pltpu.async_copy(rows0, acc_shared.at[dst_v.at[j]], ssem0,
                             add=True)
            pltpu.make_async_copy(rows0, acc_shared.at[dst_v.at[j]],
                                  ssem0).wait()

            @pl.when(j + 2 < HH)
            def _():
                pltpu.async_copy(h_hbm.at[src_v.at[j + 2]], rows0, gsem0)

            pltpu.make_async_copy(h_hbm.at[src_v.at[j + 1]], rows1,
                                  gsem1).wait()
            pltpu.async_copy(rows1, acc_shared.at[dst_v.at[j + 1]], ssem1,
                             add=True)

        pltpu.make_async_copy(rows1, acc_shared.at[dst_v.at[HH - 1]],
                              ssem1).wait()
        if half == 0:
            pltpu.sync_copy(src_hbm.at[wid, pl.ds(HH, HH)], src_v)
            pltpu.sync_copy(dst_hbm.at[wid, pl.ds(HH, HH)], dst_v)
            pltpu.async_copy(h_hbm.at[src_v.at[0]], rows0, gsem0)
            pltpu.async_copy(h_hbm.at[src_v.at[1]], rows1, gsem1)

    plsc.subcore_barrier()

    @pl.when(s < NS - 1)
    def _():
        pltpu.sync_copy(acc_shared.at[pl.ds(s * ZR, ZR)],
                        out_hbm.at[c, pl.ds(s * ZR, ZR)])

    @pl.when(s == NS - 1)
    def _():
        pltpu.sync_copy(acc_shared.at[pl.ds((NS - 1) * ZR, ZLAST)],
                        out_hbm.at[c, pl.ds((NS - 1) * ZR, ZLAST)])


RB = 2000
GB = N // RB


def _lin_kernel(h_ref, w_ref, m_ref):
    m_ref[...] = jnp.dot(h_ref[...], w_ref[...],
                         preferred_element_type=jnp.float32)


def _gru_core(part_ref, h_ref, wih_ref, whh_ref, bih_ref, bhh_ref):
    g = part_ref[0] + part_ref[1]
    gi = lax.dot_general(g, wih_ref[...], (((1,), (1,)), ((), ())),
                         preferred_element_type=jnp.float32) + bih_ref[...]
    gh = lax.dot_general(h_ref[...], whh_ref[...], (((1,), (1,)), ((), ())),
                         preferred_element_type=jnp.float32) + bhh_ref[...]
    r = jax.nn.sigmoid(gi[:, 0:D] + gh[:, 0:D])
    z = jax.nn.sigmoid(gi[:, D:2 * D] + gh[:, D:2 * D])
    n = jnp.tanh(gi[:, 2 * D:] + r * gh[:, 2 * D:])
    return (1.0 - z) * n + z * h_ref[...]


def _gru_kernel(part_ref, h_ref, wnext_ref, wih_ref, whh_ref, bih_ref,
                bhh_ref, out_ref, m_ref):
    h_new = _gru_core(part_ref, h_ref, wih_ref, whh_ref, bih_ref, bhh_ref)
    out_ref[...] = h_new
    m_ref[...] = jnp.dot(h_new, wnext_ref[...],
                         preferred_element_type=jnp.float32)


def _gru_pool_kernel(part_ref, h_ref, wih_ref, whh_ref, bih_ref,
                     bhh_ref, gw_ref, gb_ref, hg_ref, num_acc, den_acc):
    i = pl.program_id(0)
    h_new = _gru_core(part_ref, h_ref, wih_ref, whh_ref, bih_ref, bhh_ref)
    gate = jax.nn.sigmoid(
        jnp.sum(h_new * gw_ref[...], axis=1, keepdims=True) + gb_ref[...])
    e = jnp.exp(gate)
    blk_num = jnp.sum(e * h_new, axis=0, keepdims=True)
    blk_den = jnp.sum(e)

    @pl.when(i == 0)
    def _():
        num_acc[...] = blk_num
        den_acc[0, 0] = blk_den

    @pl.when(i > 0)
    def _():
        num_acc[...] += blk_num
        den_acc[0, 0] += blk_den

    @pl.when(i == GB - 1)
    def _():
        hg_ref[...] = num_acc[...] / den_acc[0, 0]


def _sc_mesh():
    return plsc.VectorSubcoreMesh(core_axis_name="c", subcore_axis_name="s")


def _embed_call(embed, xidx):
    kern = pl.kernel(
        _embed_kernel,
        out_type=jax.ShapeDtypeStruct((N, D), jnp.float32),
        mesh=_sc_mesh(),
        scratch_types=[
            pltpu.VMEM((EK, EB), jnp.int32),
            pltpu.VMEM((EB, D), jnp.float32),
            pltpu.VMEM((EB, D), jnp.float32),
            pltpu.SemaphoreType.DMA,
            pltpu.SemaphoreType.DMA,
            pltpu.SemaphoreType.DMA,
            pltpu.SemaphoreType.DMA,
        ],
    )
    return kern(embed, xidx)


def _agg_call(h, src_r, dst_r, zeros):
    kern = pl.kernel(
        _agg_kernel,
        out_type=jax.ShapeDtypeStruct((NC, N, D), jnp.float32),
        mesh=_sc_mesh(),
        scratch_types=[
            pltpu.VMEM((HH, AB), jnp.int32),
            pltpu.VMEM((HH, AB), jnp.int32),
            pltpu.VMEM((AB, D), jnp.float32),
            pltpu.VMEM((AB, D), jnp.float32),
            pltpu.SemaphoreType.DMA,
            pltpu.SemaphoreType.DMA,
            pltpu.SemaphoreType.DMA,
            pltpu.SemaphoreType.DMA,
            pltpu.VMEM_SHARED((N, D), jnp.float32),
        ],
    )
    return kern(h, src_r, dst_r, zeros)


_GRU_WEIGHT_SPECS = [
    pl.BlockSpec((3 * D, D), lambda i: (0, 0)),
    pl.BlockSpec((3 * D, D), lambda i: (0, 0)),
    pl.BlockSpec((1, 3 * D), lambda i: (0, 0)),
    pl.BlockSpec((1, 3 * D), lambda i: (0, 0)),
]


def _lin_call(h, Wi):
    return pl.pallas_call(
        _lin_kernel,
        grid=(GB,),
        in_specs=[
            pl.BlockSpec((RB, D), lambda i: (i, 0)),
            pl.BlockSpec((D, D), lambda i: (0, 0)),
        ],
        out_specs=pl.BlockSpec((RB, D), lambda i: (i, 0)),
        out_shape=jax.ShapeDtypeStruct((N, D), jnp.float32),
    )(h, Wi)


def _gru_call(part, h, Wnext, W_ih, W_hh, bih, bhh):
    return pl.pallas_call(
        _gru_kernel,
        grid=(GB,),
        in_specs=[
            pl.BlockSpec((NC, RB, D), lambda i: (0, i, 0)),
            pl.BlockSpec((RB, D), lambda i: (i, 0)),
            pl.BlockSpec((D, D), lambda i: (0, 0)),
        ] + _GRU_WEIGHT_SPECS,
        out_specs=[
            pl.BlockSpec((RB, D), lambda i: (i, 0)),
            pl.BlockSpec((RB, D), lambda i: (i, 0)),
        ],
        out_shape=[
            jax.ShapeDtypeStruct((N, D), jnp.float32),
            jax.ShapeDtypeStruct((N, D), jnp.float32),
        ],
    )(part, h, Wnext, W_ih, W_hh, bih, bhh)


def _gru_pool_call(part, h, W_ih, W_hh, bih, bhh, gw, gb):
    return pl.pallas_call(
        _gru_pool_kernel,
        grid=(GB,),
        in_specs=[
            pl.BlockSpec((NC, RB, D), lambda i: (0, i, 0)),
            pl.BlockSpec((RB, D), lambda i: (i, 0)),
        ] + _GRU_WEIGHT_SPECS + [
            pl.BlockSpec((1, D), lambda i: (0, 0)),
            pl.BlockSpec((1, 1), lambda i: (0, 0)),
        ],
        out_specs=pl.BlockSpec((1, D), lambda i: (0, 0)),
        out_shape=jax.ShapeDtypeStruct((1, D), jnp.float32),
        scratch_shapes=[
            pltpu.VMEM((1, D), jnp.float32),
            pltpu.SMEM((1, 1), jnp.float32),
        ],
    )(part, h, W_ih, W_hh, bih, bhh, gw, gb)


def kernel(x, edge_index, embed, W, W_ih, W_hh, b_ih, b_hh, gate_w, gate_b):
    xidx = jnp.concatenate(
        [x[:, 0], jnp.zeros((NPAD - N,), jnp.int32)]).reshape(NW, EK, EB)
    src_r = edge_index[0].reshape(NW, ANB, AB)
    dst_r = edge_index[1].reshape(NW, ANB, AB)
    zeros = jnp.zeros((N, D), jnp.float32)
    bih = b_ih.reshape(1, 3 * D)
    bhh = b_hh.reshape(1, 3 * D)
    gb = gate_b.reshape(1, 1)

    h = _embed_call(embed, xidx)
    m = _lin_call(h, W[0])
    for i in range(L):
        part = _agg_call(m, src_r, dst_r, zeros)
        if i < L - 1:
            h, m = _gru_call(part, h, W[i + 1], W_ih, W_hh, bih, bhh)
        else:
            hg = _gru_pool_call(part, h, W_ih, W_hh, bih, bhh, gate_w, gb)
    return hg


# speedup vs baseline: 11.4956x; 1.0234x over previous
import functools

import jax
import jax.numpy as jnp
from jax import lax
from jax.experimental import pallas as pl
from jax.experimental.pallas import tpu as pltpu
from jax.experimental.pallas import tpu_sc as plsc

N = 10000
E = 320000
D = 128
L = 5

NC = 2
NS = 16
NW = NC * NS

EB = 80
EK = 4
NPAD = NW * EK * EB

AB = 125
ANB = (E // NW) // AB
HH = ANB // 2
ZR = 632
ZLAST = N - (NS - 1) * ZR


def _embed_kernel(embed_hbm, xidx_hbm, out_hbm, idx_v, rows0, rows1,
                  gsem0, gsem1, wsem0, wsem1):
    wid = lax.axis_index("s") * NC + lax.axis_index("c")
    base = wid * EK * EB
    pltpu.sync_copy(xidx_hbm.at[wid], idx_v)
    pltpu.async_copy(embed_hbm.at[idx_v.at[0]], rows0, gsem0)

    def _wr(rows, k, wsem):
        return pltpu.make_async_copy(
            rows, out_hbm.at[pl.ds(base + k * EB, EB)], wsem)

    @pl.when(wid == NW - 1)
    def _():
        pltpu.make_async_copy(embed_hbm.at[idx_v.at[0]], rows0, gsem0).wait()
        pltpu.sync_copy(rows0, out_hbm.at[pl.ds(base, EB)])

    @pl.when(wid < NW - 1)
    def _():
        pltpu.async_copy(embed_hbm.at[idx_v.at[1]], rows1, gsem1)
        pltpu.make_async_copy(embed_hbm.at[idx_v.at[0]], rows0, gsem0).wait()
        _wr(rows0, 0, wsem0).start()
        pltpu.make_async_copy(embed_hbm.at[idx_v.at[1]], rows1, gsem1).wait()
        _wr(rows1, 1, wsem1).start()
        _wr(rows0, 0, wsem0).wait()
        pltpu.async_copy(embed_hbm.at[idx_v.at[2]], rows0, gsem0)
        _wr(rows1, 1, wsem1).wait()
        pltpu.async_copy(embed_hbm.at[idx_v.at[3]], rows1, gsem1)
        pltpu.make_async_copy(embed_hbm.at[idx_v.at[2]], rows0, gsem0).wait()
        _wr(rows0, 2, wsem0).start()
        pltpu.make_async_copy(embed_hbm.at[idx_v.at[3]], rows1, gsem1).wait()
        _wr(rows1, 3, wsem1).start()
        _wr(rows0, 2, wsem0).wait()
        _wr(rows1, 3, wsem1).wait()


def _agg_kernel(h_hbm, src_hbm, dst_hbm, out_hbm,
                src_v, dst_v, rows0, rows1, gsem0, gsem1, ssem0, ssem1,
                acc_shared):
    c = lax.axis_index("c")
    s = lax.axis_index("s")
    wid = s * NC + c
    pltpu.sync_copy(src_hbm.at[wid, pl.ds(0, HH)], src_v)
    pltpu.sync_copy(dst_hbm.at[wid, pl.ds(0, HH)], dst_v)
    pltpu.async_copy(h_hbm.at[src_v.at[0]], rows0, gsem0)

    @pl.loop(0, AB)
    def _(r):
        for cc in range(8):
            rows1[r, pl.ds(cc * 16, 16)] = jnp.zeros((16,), jnp.float32)

    base_r = s * ZR

    @pl.when(s < NS - 1)
    def _():
        for k in range(5):
            pltpu.sync_copy(rows1.at[pl.ds(0, 120)],
                            acc_shared.at[pl.ds(base_r + k * 120, 120)])
        pltpu.sync_copy(rows1.at[pl.ds(0, 32)],
                        acc_shared.at[pl.ds(base_r + 600, 32)])

    @pl.when(s == NS - 1)
    def _():
        for k in range(4):
            pltpu.sync_copy(rows1.at[pl.ds(0, 120)],
                            acc_shared.at[pl.ds(base_r + k * 120, 120)])
        pltpu.sync_copy(rows1.at[pl.ds(0, 40)],
                        acc_shared.at[pl.ds(base_r + 480, 40)])

    pltpu.async_copy(h_hbm.at[src_v.at[1]], rows1, gsem1)
    plsc.subcore_barrier()

    for half in range(2):
        @pl.loop(0, HH, step=2)
        def _(j):
            @pl.when(j > 0)
            def _():
                pltpu.make_async_copy(rows1, acc_shared.at[dst_v.at[j - 1]],
                                      ssem1).wait()
                pltpu.async_copy(h_hbm.at[src_v.at[j + 1]], rows1, gsem1)

            pltpu.make_async_copy(h_hbm.at[src_v.at[j]], rows0, gsem0).wait()
            pltpu.async_copy(rows0, acc_shared.at[dst_v.at[j]], ssem0,
                             add=True)
            pltpu.make_async_copy(rows0, acc_shared.at[dst_v.at[j]],
                                  ssem0).wait()

            @pl.when(j + 2 < HH)
            def _():
                pltpu.async_copy(h_hbm.at[src_v.at[j + 2]], rows0, gsem0)

            pltpu.make_async_copy(h_hbm.at[src_v.at[j + 1]], rows1,
                                  gsem1).wait()
            pltpu.async_copy(rows1, acc_shared.at[dst_v.at[j + 1]], ssem1,
                             add=True)

        pltpu.make_async_copy(rows1, acc_shared.at[dst_v.at[HH - 1]],
                              ssem1).wait()
        if half == 0:
            pltpu.sync_copy(src_hbm.at[wid, pl.ds(HH, HH)], src_v)
            pltpu.sync_copy(dst_hbm.at[wid, pl.ds(HH, HH)], dst_v)
            pltpu.async_copy(h_hbm.at[src_v.at[0]], rows0, gsem0)
            pltpu.async_copy(h_hbm.at[src_v.at[1]], rows1, gsem1)

    plsc.subcore_barrier()

    @pl.when(s < NS - 1)
    def _():
        pltpu.sync_copy(acc_shared.at[pl.ds(s * ZR, ZR)],
                        out_hbm.at[c, pl.ds(s * ZR, ZR)])

    @pl.when(s == NS - 1)
    def _():
        pltpu.sync_copy(acc_shared.at[pl.ds((NS - 1) * ZR, ZLAST)],
                        out_hbm.at[c, pl.ds((NS - 1) * ZR, ZLAST)])


RB = 2000
GB = N // RB


def _lin_kernel(h_ref, w_ref, m_ref):
    m_ref[...] = jnp.dot(h_ref[...], w_ref[...],
                         preferred_element_type=jnp.float32)


def _gru_core(part_ref, h_ref, wih_ref, whh_ref, bih_ref, bhh_ref):
    g = part_ref[0] + part_ref[1]
    gi = lax.dot_general(g, wih_ref[...], (((1,), (1,)), ((), ())),
                         preferred_element_type=jnp.float32) + bih_ref[...]
    gh = lax.dot_general(h_ref[...], whh_ref[...], (((1,), (1,)), ((), ())),
                         preferred_element_type=jnp.float32) + bhh_ref[...]
    r = jax.nn.sigmoid(gi[:, 0:D] + gh[:, 0:D])
    z = jax.nn.sigmoid(gi[:, D:2 * D] + gh[:, D:2 * D])
    n = jnp.tanh(gi[:, 2 * D:] + r * gh[:, 2 * D:])
    return (1.0 - z) * n + z * h_ref[...]


def _gru_kernel(part_ref, h_ref, wnext_ref, wih_ref, whh_ref, bih_ref,
                bhh_ref, out_ref, m_ref):
    h_new = _gru_core(part_ref, h_ref, wih_ref, whh_ref, bih_ref, bhh_ref)
    out_ref[...] = h_new
    m_ref[...] = jnp.dot(h_new, wnext_ref[...],
                         preferred_element_type=jnp.float32)


def _gru_pool_kernel(part_ref, h_ref, wih_ref, whh_ref, bih_ref,
                     bhh_ref, gw_ref, gb_ref, hg_ref, num_acc, den_acc):
    i = pl.program_id(0)
    h_new = _gru_core(part_ref, h_ref, wih_ref, whh_ref, bih_ref, bhh_ref)
    gate = jax.nn.sigmoid(
        jnp.sum(h_new * gw_ref[...], axis=1, keepdims=True) + gb_ref[...])
    e = jnp.exp(gate)
    blk_num = jnp.sum(e * h_new, axis=0, keepdims=True)
    blk_den = jnp.sum(e)

    @pl.when(i == 0)
    def _():
        num_acc[...] = blk_num
        den_acc[0, 0] = blk_den

    @pl.when(i > 0)
    def _():
        num_acc[...] += blk_num
        den_acc[0, 0] += blk_den

    @pl.when(i == GB - 1)
    def _():
        hg_ref[...] = num_acc[...] / den_acc[0, 0]


def _sc_mesh():
    return plsc.VectorSubcoreMesh(core_axis_name="c", subcore_axis_name="s")


def _embed_call(embed, xidx):
    kern = pl.kernel(
        _embed_kernel,
        out_type=jax.ShapeDtypeStruct((N, D), jnp.float32),
        mesh=_sc_mesh(),
        scratch_types=[
            pltpu.VMEM((EK, EB), jnp.int32),
            pltpu.VMEM((EB, D), jnp.float32),
            pltpu.VMEM((EB, D), jnp.float32),
            pltpu.SemaphoreType.DMA,
            pltpu.SemaphoreType.DMA,
            pltpu.SemaphoreType.DMA,
            pltpu.SemaphoreType.DMA,
        ],
    )
    return kern(embed, xidx)


def _agg_call(h, src_r, dst_r):
    kern = pl.kernel(
        _agg_kernel,
        out_type=jax.ShapeDtypeStruct((NC, N, D), jnp.float32),
        mesh=_sc_mesh(),
        scratch_types=[
            pltpu.VMEM((HH, AB), jnp.int32),
            pltpu.VMEM((HH, AB), jnp.int32),
            pltpu.VMEM((AB, D), jnp.float32),
            pltpu.VMEM((AB, D), jnp.float32),
            pltpu.SemaphoreType.DMA,
            pltpu.SemaphoreType.DMA,
            pltpu.SemaphoreType.DMA,
            pltpu.SemaphoreType.DMA,
            pltpu.VMEM_SHARED((N, D), jnp.float32),
        ],
    )
    return kern(h, src_r, dst_r)


_GRU_WEIGHT_SPECS = [
    pl.BlockSpec((3 * D, D), lambda i: (0, 0)),
    pl.BlockSpec((3 * D, D), lambda i: (0, 0)),
    pl.BlockSpec((1, 3 * D), lambda i: (0, 0)),
    pl.BlockSpec((1, 3 * D), lambda i: (0, 0)),
]


def _lin_call(h, Wi):
    return pl.pallas_call(
        _lin_kernel,
        grid=(GB,),
        in_specs=[
            pl.BlockSpec((RB, D), lambda i: (i, 0)),
            pl.BlockSpec((D, D), lambda i: (0, 0)),
        ],
        out_specs=pl.BlockSpec((RB, D), lambda i: (i, 0)),
        out_shape=jax.ShapeDtypeStruct((N, D), jnp.float32),
    )(h, Wi)


def _gru_call(part, h, Wnext, W_ih, W_hh, bih, bhh):
    return pl.pallas_call(
        _gru_kernel,
        grid=(GB,),
        in_specs=[
            pl.BlockSpec((NC, RB, D), lambda i: (0, i, 0)),
            pl.BlockSpec((RB, D), lambda i: (i, 0)),
            pl.BlockSpec((D, D), lambda i: (0, 0)),
        ] + _GRU_WEIGHT_SPECS,
        out_specs=[
            pl.BlockSpec((RB, D), lambda i: (i, 0)),
            pl.BlockSpec((RB, D), lambda i: (i, 0)),
        ],
        out_shape=[
            jax.ShapeDtypeStruct((N, D), jnp.float32),
            jax.ShapeDtypeStruct((N, D), jnp.float32),
        ],
    )(part, h, Wnext, W_ih, W_hh, bih, bhh)


def _gru_pool_call(part, h, W_ih, W_hh, bih, bhh, gw, gb):
    return pl.pallas_call(
        _gru_pool_kernel,
        grid=(GB,),
        in_specs=[
            pl.BlockSpec((NC, RB, D), lambda i: (0, i, 0)),
            pl.BlockSpec((RB, D), lambda i: (i, 0)),
        ] + _GRU_WEIGHT_SPECS + [
            pl.BlockSpec((1, D), lambda i: (0, 0)),
            pl.BlockSpec((1, 1), lambda i: (0, 0)),
        ],
        out_specs=pl.BlockSpec((1, D), lambda i: (0, 0)),
        out_shape=jax.ShapeDtypeStruct((1, D), jnp.float32),
        scratch_shapes=[
            pltpu.VMEM((1, D), jnp.float32),
            pltpu.SMEM((1, 1), jnp.float32),
        ],
    )(part, h, W_ih, W_hh, bih, bhh, gw, gb)


def kernel(x, edge_index, embed, W, W_ih, W_hh, b_ih, b_hh, gate_w, gate_b):
    xidx = jnp.concatenate(
        [x[:, 0], jnp.zeros((NPAD - N,), jnp.int32)]).reshape(NW, EK, EB)
    src_r = edge_index[0].reshape(NW, ANB, AB)
    dst_r = edge_index[1].reshape(NW, ANB, AB)
    bih = b_ih.reshape(1, 3 * D)
    bhh = b_hh.reshape(1, 3 * D)
    gb = gate_b.reshape(1, 1)

    h = _embed_call(embed, xidx)
    m = _lin_call(h, W[0])
    for i in range(L):
        part = _agg_call(m, src_r, dst_r)
        if i < L - 1:
            h, m = _gru_call(part, h, W[i + 1], W_ih, W_hh, bih, bhh)
        else:
            hg = _gru_pool_call(part, h, W_ih, W_hh, bih, bhh, gate_w, gb)
    return hg
